# baseline (speedup 1.0000x reference)
import functools

import jax
import jax.numpy as jnp
from jax import lax
from jax.experimental import pallas as pl
from jax.experimental.pallas import tpu as pltpu
from jax.experimental.pallas import tpu_sc as plsc

EMBED = 64
OUT_D = 65
LANES = 16
NC, NS = 2, 16
NW = NC * NS
CH = 256
K = 16


def _emb_body(bpw, pitch_hbm, idx_hbm, table_hbm, out_hbm,
              idx_v, pitch_v, bufs, out_v, sems, isem):
    wid = lax.axis_index("s") * NC + lax.axis_index("c")
    base = wid * bpw
    iota = lax.iota(jnp.int32, LANES)

    def chunk_body(ck, carry):
        cbase = base + ck * CH
        pltpu.async_copy(idx_hbm.at[pl.ds(cbase, CH)], idx_v, isem).wait()
        pltpu.sync_copy(pitch_hbm.at[pl.ds(cbase, CH)], pitch_v)

        def group_body(g, c2):
            vec = idx_v[pl.ds(g * LANES, LANES)]
            blk = (vec // 8) * 8
            rin_vec = vec % 8
            for l in range(K):
                ab = pl.multiple_of(blk[l], 8)
                pltpu.async_copy(
                    table_hbm.at[pl.ds(ab, 8)], bufs.at[l], sems.at[l]
                )
            for l in range(K):
                pltpu.make_async_copy(
                    table_hbm.at[pl.ds(0, 8)], bufs.at[l], sems.at[l]
                ).wait()
                r = g * LANES + l
                rin = rin_vec[l]
                for c in range(EMBED // LANES):
                    out_v[r, pl.ds(1 + c * LANES, LANES)] = (
                        bufs[l, rin, pl.ds(c * LANES, LANES)]
                    )
            return c2

        lax.fori_loop(0, CH // LANES, group_body, 0)

        zeros = jnp.zeros((LANES,), jnp.int32)

        def pitch_body(g, c2):
            vals = pitch_v[pl.ds(g * LANES, LANES)]
            plsc.store_scatter(out_v, [iota + g * LANES, zeros], vals)
            return c2

        lax.fori_loop(0, CH // LANES, pitch_body, 0)

        pltpu.sync_copy(out_v, out_hbm.at[pl.ds(cbase, CH)])
        return carry

    lax.fori_loop(0, bpw // CH, chunk_body, 0)


def kernel(pitch, timbre_id, table):
    batch = pitch.shape[0]
    bpw = batch // NW

    mesh = plsc.VectorSubcoreMesh(
        core_axis_name="c", subcore_axis_name="s", num_cores=NC, num_subcores=NS
    )
    run = functools.partial(
        pl.kernel,
        out_type=jax.ShapeDtypeStruct((batch, OUT_D), jnp.float32),
        mesh=mesh,
        compiler_params=pltpu.CompilerParams(
            needs_layout_passes=False, use_tc_tiling_on_sc=True
        ),
        scratch_types=[
            pltpu.VMEM((CH,), jnp.int32),
            pltpu.VMEM((CH,), jnp.float32),
            pltpu.VMEM((K, 8, EMBED), jnp.float32),
            pltpu.VMEM((CH, OUT_D), jnp.float32),
            pltpu.SemaphoreType.DMA((K,)),
            pltpu.SemaphoreType.DMA,
        ],
    )(functools.partial(_emb_body, bpw))
    return run(pitch, timbre_id, table)


# speedup vs baseline: 1.4353x; 1.4353x over previous
import functools

import jax
import jax.numpy as jnp
from jax import lax
from jax.experimental import pallas as pl
from jax.experimental.pallas import tpu as pltpu
from jax.experimental.pallas import tpu_sc as plsc

EMBED = 64
PADW = 128
OUT_D = 65
LANES = 16
NC, NS = 2, 16
NW = NC * NS
CH = 256
NB = 2
UNROLL = 8


def _emb_body(bpw, pitch_hbm, idx_hbm, table_hbm, out_hbm,
              idx_v, pitch_v, rows_v, out_v, gsems):
    wid = lax.axis_index("s") * NC + lax.axis_index("c")
    base = wid * bpw
    iota = lax.iota(jnp.int32, LANES)
    nch = bpw // CH

    pltpu.sync_copy(idx_hbm.at[pl.ds(base, bpw)], idx_v)
    pltpu.sync_copy(pitch_hbm.at[pl.ds(base, bpw)], pitch_v)

    def gather(k):
        pltpu.async_copy(
            table_hbm.at[idx_v.at[pl.ds(k * CH, CH)]], rows_v.at[k % NB],
            gsems.at[k % NB],
        )

    gather(0)
    for k in range(nch):
        b = k % NB
        pltpu.make_async_copy(
            table_hbm.at[pl.ds(0, CH)], rows_v.at[b], gsems.at[b]
        ).wait()
        if k + 1 < nch:
            gather(k + 1)

        def row_body(g, c2, b=b):
            for u in range(UNROLL):
                r = g * UNROLL + u
                for c in range(EMBED // LANES):
                    out_v[r, pl.ds(1 + c * LANES, LANES)] = (
                        rows_v[b, r, pl.ds(c * LANES, LANES)]
                    )
            return c2

        lax.fori_loop(0, CH // UNROLL, row_body, 0)

        zsplat = jnp.zeros((LANES,), jnp.int32)

        def pitch_body(g, c2, k=k, zsplat=zsplat):
            vals = pitch_v[pl.ds(k * CH + g * LANES, LANES)]
            ridx = iota + g * LANES
            plsc.store_scatter(out_v, [ridx, zsplat], vals)
            return c2

        lax.fori_loop(0, CH // LANES, pitch_body, 0)

        pltpu.sync_copy(out_v, out_hbm.at[pl.ds(base + k * CH, CH)])


def kernel(pitch, timbre_id, table):
    batch = pitch.shape[0]
    bpw = batch // NW

    table_p = jnp.pad(table, ((0, 0), (0, PADW - EMBED)))

    mesh = plsc.VectorSubcoreMesh(
        core_axis_name="c", subcore_axis_name="s", num_cores=NC, num_subcores=NS
    )
    run = functools.partial(
        pl.kernel,
        out_type=jax.ShapeDtypeStruct((batch, OUT_D), jnp.float32),
        mesh=mesh,
        compiler_params=pltpu.CompilerParams(
            needs_layout_passes=False, use_tc_tiling_on_sc=True
        ),
        scratch_types=[
            pltpu.VMEM((batch // NW,), jnp.int32),
            pltpu.VMEM((batch // NW,), jnp.float32),
            pltpu.VMEM((NB, CH, PADW), jnp.float32),
            pltpu.VMEM((CH, OUT_D), jnp.float32),
            pltpu.SemaphoreType.DMA((NB,)),
        ],
    )(functools.partial(_emb_body, bpw))
    return run(pitch, timbre_id, table_p)
